# Initial kernel scaffold; baseline (speedup 1.0000x reference)
import jax
import jax.numpy as jnp
from jax.experimental import pallas as pl


def kernel(x, Wr, br, W_in, b_in, W_out, b_out):
    raise NotImplementedError("write your pallas kernel here")


# speedup vs baseline: 1.5327x; 1.5327x over previous
import functools

import jax
import jax.numpy as jnp
from jax import lax
from jax.experimental import pallas as pl
from jax.experimental.pallas import tpu as pltpu
from jax.experimental.pallas import tpu_sc as plsc

N = 2048
H = 1024
I = 2048
E = 8
EP = 128
BT = 256
G = 24
P = G * BT
NW = 32
TPW = N // NW
CPW = 32

NEG = -3e38


def _route_body(x_ref, wr_ref, nb_ref, pos_ref, w1_ref, w2_ref, meta_ref):
    x = x_ref[...]
    wr = wr_ref[...]
    y = lax.dot_general(x, wr, (((1,), (1,)), ((), ())),
                        preferred_element_type=jnp.float32)
    y = y + nb_ref[...]
    lane = lax.broadcasted_iota(jnp.int32, (N, EP), 1)
    m1 = jnp.max(y, axis=1, keepdims=True)
    i1 = jnp.min(jnp.where(y == m1, lane, EP + 1), axis=1, keepdims=True)
    y2 = jnp.where(lane == i1, NEG, y)
    m2 = jnp.max(y2, axis=1, keepdims=True)
    i2 = jnp.min(jnp.where(y2 == m2, lane, EP + 1), axis=1, keepdims=True)
    e2 = jnp.exp(m2 - m1)
    w1 = 1.0 / (1.0 + e2)
    w2 = e2 / (1.0 + e2)

    oh1 = (lane == i1).astype(jnp.float32)
    oh2 = (lane == i2).astype(jnp.float32)
    assign = oh1 + oh2
    r = lax.broadcasted_iota(jnp.int32, (N, N), 0)
    c = lax.broadcasted_iota(jnp.int32, (N, N), 1)
    lt = (r >= c).astype(jnp.float32)
    cinc = lax.dot_general(lt, assign, (((1,), (0,)), ((), ())),
                           preferred_element_type=jnp.float32)
    cexc = cinc - assign
    cnt = cinc[N - 1:N, :]
    cnt_i = cnt.astype(jnp.int32)
    pc = ((cnt_i + (BT - 1)) // BT) * BT
    a = lax.broadcasted_iota(jnp.int32, (EP, EP), 0)
    b = lax.broadcasted_iota(jnp.int32, (EP, EP), 1)
    sl = (a < b).astype(jnp.float32)
    off = lax.dot_general(pc.astype(jnp.float32), sl, (((1,), (0,)), ((), ())),
                          preferred_element_type=jnp.float32)
    off_i = off.astype(jnp.int32)

    posf = off + cexc
    pos1 = jnp.sum(oh1 * posf, axis=1, keepdims=True).astype(jnp.int32)
    pos2 = jnp.sum(oh2 * posf, axis=1, keepdims=True).astype(jnp.int32)
    pos_ref[...] = ((lane == 0) * pos1 + (lane == 1) * pos2).astype(jnp.int32)
    w1_ref[...] = jnp.broadcast_to(w1, (N, EP))
    w2_ref[...] = jnp.broadcast_to(w2, (N, EP))

    gi = lax.broadcasted_iota(jnp.int32, (1, EP), 1)
    gstart = gi * BT
    ends = off_i + pc
    gid = jnp.zeros((1, EP), jnp.int32)
    for e in range(E):
        gid = gid + (gstart >= ends[:, e:e + 1]).astype(jnp.int32)
    gid = jnp.minimum(gid, E - 1)
    active = (gstart < ends[:, E - 1:E]).astype(jnp.int32)
    row = lax.broadcasted_iota(jnp.int32, (8, EP), 0)
    meta = jnp.where(row == 0, jnp.broadcast_to(gid, (8, EP)),
                     jnp.where(row == 1, jnp.broadcast_to(active, (8, EP)), 0))
    meta_ref[...] = meta.astype(jnp.int32)


def _route_call(x, wr_pad, nb):
    return pl.pallas_call(
        _route_body,
        out_shape=(
            jax.ShapeDtypeStruct((N, EP), jnp.int32),
            jax.ShapeDtypeStruct((N, EP), jnp.float32),
            jax.ShapeDtypeStruct((N, EP), jnp.float32),
            jax.ShapeDtypeStruct((8, EP), jnp.int32),
        ),
    )(x, wr_pad, nb)


def _sc_mesh():
    return plsc.VectorSubcoreMesh(core_axis_name="c", subcore_axis_name="s",
                                  num_cores=2, num_subcores=16)


def _scatter_body(x_hbm, w1_hbm, w2_hbm, idx0_hbm, idx1_hbm,
                  xs_hbm, ws_hbm, xrows, w1rows, w2rows, i0v, i1v, sem):
    wid = lax.axis_index("s") * 2 + lax.axis_index("c")
    base = wid * TPW
    pltpu.sync_copy(idx0_hbm.at[wid], i0v)
    pltpu.sync_copy(idx1_hbm.at[wid], i1v)
    pltpu.sync_copy(x_hbm.at[pl.ds(base, TPW)], xrows)
    pltpu.sync_copy(w1_hbm.at[pl.ds(base, TPW)], w1rows)
    pltpu.sync_copy(w2_hbm.at[pl.ds(base, TPW)], w2rows)
    pltpu.async_copy(xrows, xs_hbm.at[i0v], sem).wait()
    pltpu.async_copy(xrows, xs_hbm.at[i1v], sem).wait()
    pltpu.async_copy(w1rows, ws_hbm.at[i0v], sem).wait()
    pltpu.async_copy(w2rows, ws_hbm.at[i1v], sem).wait()


@functools.lru_cache(maxsize=None)
def _scatter_kernel():
    return pl.kernel(
        _scatter_body,
        out_type=(
            jax.ShapeDtypeStruct((P, H), jnp.float32),
            jax.ShapeDtypeStruct((P, EP), jnp.float32),
        ),
        mesh=_sc_mesh(),
        scratch_types=[
            pltpu.VMEM((TPW, H), jnp.float32),
            pltpu.VMEM((TPW, EP), jnp.float32),
            pltpu.VMEM((TPW, EP), jnp.float32),
            pltpu.VMEM((TPW,), jnp.int32),
            pltpu.VMEM((TPW,), jnp.int32),
            pltpu.SemaphoreType.DMA,
        ],
    )


def _ffn_body(gid_ref, act_ref, xs_ref, ws_ref, win_ref, bin_ref,
              wout_ref, bout_ref, y_ref):
    g = pl.program_id(0)

    @pl.when(act_ref[g] == 1)
    def _():
        xb = xs_ref[...]
        win = win_ref[0]
        h = lax.dot_general(xb, win, (((1,), (1,)), ((), ())),
                            preferred_element_type=jnp.float32)
        h = h + bin_ref[0]
        x1 = h[:, :I]
        x2 = h[:, I:]
        act_v = x1 * (1.0 / (1.0 + jnp.exp(-x2)))
        wout = wout_ref[0]
        eo = lax.dot_general(act_v, wout, (((1,), (1,)), ((), ())),
                             preferred_element_type=jnp.float32)
        eo = eo + bout_ref[0]
        y_ref[...] = eo * ws_ref[:, 0:1]


def _ffn_call(gids, act, xs, ws, W_in, b_in, W_out, b_out):
    grid_spec = pltpu.PrefetchScalarGridSpec(
        num_scalar_prefetch=2,
        grid=(G,),
        in_specs=[
            pl.BlockSpec((BT, H), lambda g, gid, a: (g, 0)),
            pl.BlockSpec((BT, EP), lambda g, gid, a: (g, 0)),
            pl.BlockSpec((1, 2 * I, H), lambda g, gid, a: (gid[g], 0, 0)),
            pl.BlockSpec((1, 1, 2 * I), lambda g, gid, a: (gid[g], 0, 0)),
            pl.BlockSpec((1, H, I), lambda g, gid, a: (gid[g], 0, 0)),
            pl.BlockSpec((1, 1, H), lambda g, gid, a: (gid[g], 0, 0)),
        ],
        out_specs=pl.BlockSpec((BT, H), lambda g, gid, a: (g, 0)),
    )
    return pl.pallas_call(
        _ffn_body,
        grid_spec=grid_spec,
        out_shape=jax.ShapeDtypeStruct((P, H), jnp.float32),
        compiler_params=pltpu.CompilerParams(
            dimension_semantics=("arbitrary",)),
    )(gids, act, xs, ws, W_in, b_in.reshape(E, 1, 2 * I),
      W_out, b_out.reshape(E, 1, H))


def _combine_body(y_hbm, g0_hbm, g1_hbm, out_hbm, rows0, rows1, i0v, i1v, sem):
    wid = lax.axis_index("s") * 2 + lax.axis_index("c")
    for sub in range(TPW // CPW):
        row = wid * (TPW // CPW) + sub
        base = wid * TPW + sub * CPW
        pltpu.sync_copy(g0_hbm.at[row], i0v)
        pltpu.sync_copy(g1_hbm.at[row], i1v)
        pltpu.async_copy(y_hbm.at[i0v], rows0, sem).wait()
        pltpu.async_copy(y_hbm.at[i1v], rows1, sem).wait()
        for t in range(CPW):
            def body(j, _, t=t):
                o = j * 16
                rows0[t, pl.ds(o, 16)] = (rows0[t, pl.ds(o, 16)] +
                                          rows1[t, pl.ds(o, 16)])
                return 0
            lax.fori_loop(0, H // 16, body, 0)
        pltpu.sync_copy(rows0, out_hbm.at[pl.ds(base, CPW)])


@functools.lru_cache(maxsize=None)
def _combine_kernel():
    return pl.kernel(
        _combine_body,
        out_type=jax.ShapeDtypeStruct((N, H), jnp.float32),
        mesh=_sc_mesh(),
        scratch_types=[
            pltpu.VMEM((CPW, H), jnp.float32),
            pltpu.VMEM((CPW, H), jnp.float32),
            pltpu.VMEM((CPW,), jnp.int32),
            pltpu.VMEM((CPW,), jnp.int32),
            pltpu.SemaphoreType.DMA,
        ],
    )


def kernel(x, Wr, br, W_in, b_in, W_out, b_out):
    u = jax.random.uniform(jax.random.key(42), (N, E), dtype=jnp.float32)
    noise = -jnp.log(-jnp.log(u + 1e-20) + 1e-20)
    nb = jnp.full((N, EP), NEG, jnp.float32).at[:, :E].set(noise + br[None, :])
    wr_pad = jnp.zeros((EP, H), jnp.float32).at[:E, :].set(Wr)

    pos, w1f, w2f, meta = _route_call(x, wr_pad, nb)
    pos0 = pos[:, 0]
    pos1 = pos[:, 1]

    xs, ws = _scatter_kernel()(x, w1f, w2f,
                               pos0.reshape(NW, TPW), pos1.reshape(NW, TPW))

    gids = meta[0, :G]
    act = meta[1, :G]
    y = _ffn_call(gids, act, xs, ws, W_in, b_in, W_out, b_out)

    out = _combine_kernel()(y, pos0.reshape(N // CPW, CPW),
                            pos1.reshape(N // CPW, CPW))
    return out


# speedup vs baseline: 1.7190x; 1.1215x over previous
import functools

import jax
import jax.numpy as jnp
from jax import lax
from jax.experimental import pallas as pl
from jax.experimental.pallas import tpu as pltpu
from jax.experimental.pallas import tpu_sc as plsc

N = 2048
H = 1024
I = 2048
E = 8
EP = 128
BT = 256
G = 24
P = G * BT
NW = 32
TPW = N // NW
CPW = 16

NEG = -3e38


def _route_body(x_ref, wr_ref, nb_ref, br_ref, pos_ref, w1_ref, w2_ref, meta_ref):
    x = x_ref[...]
    wr = wr_ref[...]
    y = lax.dot_general(x, wr, (((1,), (1,)), ((), ())),
                        preferred_element_type=jnp.float32)
    y = y + nb_ref[...] + br_ref[...]
    lane = lax.broadcasted_iota(jnp.int32, (N, E), 1)
    m1 = jnp.max(y, axis=1, keepdims=True)
    i1 = jnp.min(jnp.where(y == m1, lane, E + 1), axis=1, keepdims=True)
    y2 = jnp.where(lane == i1, NEG, y)
    m2 = jnp.max(y2, axis=1, keepdims=True)
    i2 = jnp.min(jnp.where(y2 == m2, lane, E + 1), axis=1, keepdims=True)
    e2 = jnp.exp(m2 - m1)
    w1 = 1.0 / (1.0 + e2)
    w2 = e2 / (1.0 + e2)

    oh1 = (lane == i1).astype(jnp.float32)
    oh2 = (lane == i2).astype(jnp.float32)
    assign = oh1 + oh2
    r = lax.broadcasted_iota(jnp.int32, (N, N), 0)
    c = lax.broadcasted_iota(jnp.int32, (N, N), 1)
    lt = (r >= c).astype(jnp.float32)
    cinc = lax.dot_general(lt, assign, (((1,), (0,)), ((), ())),
                           preferred_element_type=jnp.float32)
    cexc = cinc - assign
    cnt = cinc[N - 1:N, :]
    cnt_i = cnt.astype(jnp.int32)
    pc = ((cnt_i + (BT - 1)) // BT) * BT
    a = lax.broadcasted_iota(jnp.int32, (E, E), 0)
    b = lax.broadcasted_iota(jnp.int32, (E, E), 1)
    sl = (a < b).astype(jnp.float32)
    off = lax.dot_general(pc.astype(jnp.float32), sl, (((1,), (0,)), ((), ())),
                          preferred_element_type=jnp.float32)
    off_i = off.astype(jnp.int32)

    posf = off + cexc
    pos1 = jnp.sum(oh1 * posf, axis=1, keepdims=True).astype(jnp.int32)
    pos2 = jnp.sum(oh2 * posf, axis=1, keepdims=True).astype(jnp.int32)
    pos_ref[...] = ((lane == 0) * pos1 + (lane == 1) * pos2).astype(jnp.int32)
    w1_ref[...] = jnp.broadcast_to(w1, (N, EP))
    w2_ref[...] = jnp.broadcast_to(w2, (N, EP))

    gi = lax.broadcasted_iota(jnp.int32, (1, EP), 1)
    gstart = gi * BT
    ends = off_i + pc
    gid = jnp.zeros((1, EP), jnp.int32)
    for e in range(E):
        gid = gid + (gstart >= ends[:, e:e + 1]).astype(jnp.int32)
    gid = jnp.minimum(gid, E - 1)
    active = (gstart < ends[:, E - 1:E]).astype(jnp.int32)
    row = lax.broadcasted_iota(jnp.int32, (8, EP), 0)
    meta = jnp.where(row == 0, jnp.broadcast_to(gid, (8, EP)),
                     jnp.where(row == 1, jnp.broadcast_to(active, (8, EP)), 0))
    meta_ref[...] = meta.astype(jnp.int32)


def _route_call(x, Wr, nb, br):
    return pl.pallas_call(
        _route_body,
        out_shape=(
            jax.ShapeDtypeStruct((N, E), jnp.int32),
            jax.ShapeDtypeStruct((N, EP), jnp.float32),
            jax.ShapeDtypeStruct((N, EP), jnp.float32),
            jax.ShapeDtypeStruct((8, EP), jnp.int32),
        ),
    )(x, Wr, nb, br)


def _sc_mesh():
    return plsc.VectorSubcoreMesh(core_axis_name="c", subcore_axis_name="s",
                                  num_cores=2, num_subcores=16)


def _scatter_body(x_hbm, w1_hbm, w2_hbm, idx0_hbm, idx1_hbm,
                  xs_hbm, ws_hbm, xrows, w1rows, w2rows, i0v, i1v,
                  sa, sb, sc, sd, se):
    wid = lax.axis_index("s") * 2 + lax.axis_index("c")
    base = wid * TPW
    c_i0 = pltpu.async_copy(idx0_hbm.at[wid], i0v, sa)
    c_i1 = pltpu.async_copy(idx1_hbm.at[wid], i1v, sb)
    c_x = pltpu.async_copy(x_hbm.at[pl.ds(base, TPW)], xrows, sc)
    c_w1 = pltpu.async_copy(w1_hbm.at[pl.ds(base, TPW)], w1rows, sd)
    c_w2 = pltpu.async_copy(w2_hbm.at[pl.ds(base, TPW)], w2rows, se)
    c_i0.wait()
    c_i1.wait()
    c_x.wait()
    s0 = pltpu.async_copy(xrows, xs_hbm.at[i0v], sa)
    s1 = pltpu.async_copy(xrows, xs_hbm.at[i1v], sb)
    c_w1.wait()
    s2 = pltpu.async_copy(w1rows, ws_hbm.at[i0v], sc)
    c_w2.wait()
    s3 = pltpu.async_copy(w2rows, ws_hbm.at[i1v], sd)
    s0.wait()
    s1.wait()
    s2.wait()
    s3.wait()


@functools.lru_cache(maxsize=None)
def _scatter_kernel():
    return pl.kernel(
        _scatter_body,
        out_type=(
            jax.ShapeDtypeStruct((P, H), jnp.float32),
            jax.ShapeDtypeStruct((P, EP), jnp.float32),
        ),
        mesh=_sc_mesh(),
        scratch_types=[
            pltpu.VMEM((TPW, H), jnp.float32),
            pltpu.VMEM((TPW, EP), jnp.float32),
            pltpu.VMEM((TPW, EP), jnp.float32),
            pltpu.VMEM((TPW,), jnp.int32),
            pltpu.VMEM((TPW,), jnp.int32),
            pltpu.SemaphoreType.DMA,
            pltpu.SemaphoreType.DMA,
            pltpu.SemaphoreType.DMA,
            pltpu.SemaphoreType.DMA,
            pltpu.SemaphoreType.DMA,
        ],
    )


HH = H // 2


def _ffn_body(gid_ref, act_ref, xs_ref, ws_ref, win1_ref, win2_ref, bin_ref,
              wouta_ref, woutb_ref, bout_ref, y_ref):
    g = pl.program_id(0)

    @pl.when(act_ref[g] == 1)
    def _():
        xb = xs_ref[...]
        h1 = lax.dot_general(xb, win1_ref[0], (((1,), (1,)), ((), ())),
                             preferred_element_type=jnp.float32)
        h2 = lax.dot_general(xb, win2_ref[0], (((1,), (1,)), ((), ())),
                             preferred_element_type=jnp.float32)
        h1 = h1 + bin_ref[0][:, :I]
        h2 = h2 + bin_ref[0][:, I:]
        a = h1 * (1.0 / (1.0 + jnp.exp(-h2)))
        w = ws_ref[:, 0:1]
        eoa = lax.dot_general(a, wouta_ref[0], (((1,), (1,)), ((), ())),
                              preferred_element_type=jnp.float32)
        y_ref[:, :HH] = (eoa + bout_ref[0][:, :HH]) * w
        eob = lax.dot_general(a, woutb_ref[0], (((1,), (1,)), ((), ())),
                              preferred_element_type=jnp.float32)
        y_ref[:, HH:] = (eob + bout_ref[0][:, HH:]) * w


def _ffn_call(gids, act, xs, ws, W_in, b_in, W_out, b_out):
    grid_spec = pltpu.PrefetchScalarGridSpec(
        num_scalar_prefetch=2,
        grid=(G,),
        in_specs=[
            pl.BlockSpec((BT, H), lambda g, gid, a: (g, 0)),
            pl.BlockSpec((BT, EP), lambda g, gid, a: (g, 0)),
            pl.BlockSpec((1, I, H), lambda g, gid, a: (gid[g], 0, 0)),
            pl.BlockSpec((1, I, H), lambda g, gid, a: (gid[g], 1, 0)),
            pl.BlockSpec((1, 1, 2 * I), lambda g, gid, a: (gid[g], 0, 0)),
            pl.BlockSpec((1, HH, I), lambda g, gid, a: (gid[g], 0, 0)),
            pl.BlockSpec((1, HH, I), lambda g, gid, a: (gid[g], 1, 0)),
            pl.BlockSpec((1, 1, H), lambda g, gid, a: (gid[g], 0, 0)),
        ],
        out_specs=pl.BlockSpec((BT, H), lambda g, gid, a: (g, 0)),
    )
    return pl.pallas_call(
        _ffn_body,
        grid_spec=grid_spec,
        out_shape=jax.ShapeDtypeStruct((P, H), jnp.float32),
        compiler_params=pltpu.CompilerParams(
            dimension_semantics=("arbitrary",),
            vmem_limit_bytes=120 * 1024 * 1024),
    )(gids, act, xs, ws, W_in, W_in, b_in.reshape(E, 1, 2 * I),
      W_out, W_out, b_out.reshape(E, 1, H))


def _combine_body(y_hbm, g0_hbm, g1_hbm, out_hbm,
                  i0v, i1v, r0a, r1a, r0b, r1b, sa, sb, so0, so1):
    wid = lax.axis_index("s") * 2 + lax.axis_index("c")
    ci0 = pltpu.async_copy(g0_hbm.at[wid], i0v, sa)
    ci1 = pltpu.async_copy(g1_hbm.at[wid], i1v, sb)
    ci0.wait()
    ci1.wait()
    r0 = (r0a, r0b)
    r1 = (r1a, r1b)
    so = (so0, so1)
    pending = [None, None]
    for sub in range(TPW // CPW):
        s = sub % 2
        if pending[s] is not None:
            pending[s].wait()
        g0 = pltpu.async_copy(y_hbm.at[i0v.at[pl.ds(sub * CPW, CPW)]], r0[s], sa)
        g1 = pltpu.async_copy(y_hbm.at[i1v.at[pl.ds(sub * CPW, CPW)]], r1[s], sb)
        g0.wait()
        g1.wait()
        def body(i, _, rr0=r0[s], rr1=r1[s]):
            t = i // (H // 16)
            o = (i % (H // 16)) * 16
            rr0[t, pl.ds(o, 16)] = (rr0[t, pl.ds(o, 16)] +
                                    rr1[t, pl.ds(o, 16)])
            return 0
        lax.fori_loop(0, CPW * (H // 16), body, 0, unroll=8)
        pending[s] = pltpu.async_copy(
            r0[s], out_hbm.at[pl.ds(wid * TPW + sub * CPW, CPW)], so[s])
    for p in pending:
        if p is not None:
            p.wait()


@functools.lru_cache(maxsize=None)
def _combine_kernel():
    return pl.kernel(
        _combine_body,
        out_type=jax.ShapeDtypeStruct((N, H), jnp.float32),
        mesh=_sc_mesh(),
        scratch_types=[
            pltpu.VMEM((TPW,), jnp.int32),
            pltpu.VMEM((TPW,), jnp.int32),
            pltpu.VMEM((CPW, H), jnp.float32),
            pltpu.VMEM((CPW, H), jnp.float32),
            pltpu.VMEM((CPW, H), jnp.float32),
            pltpu.VMEM((CPW, H), jnp.float32),
            pltpu.SemaphoreType.DMA,
            pltpu.SemaphoreType.DMA,
            pltpu.SemaphoreType.DMA,
            pltpu.SemaphoreType.DMA,
        ],
    )


def kernel(x, Wr, br, W_in, b_in, W_out, b_out):
    u = jax.random.uniform(jax.random.key(42), (N, E), dtype=jnp.float32)
    noise = -jnp.log(-jnp.log(u + 1e-20) + 1e-20)

    pos, w1f, w2f, meta = _route_call(x, Wr, noise, br.reshape(1, E))
    pos0 = pos[:, 0].reshape(NW, TPW)
    pos1 = pos[:, 1].reshape(NW, TPW)

    xs, ws = _scatter_kernel()(x, w1f, w2f, pos0, pos1)
    gids = meta[0, :G]
    act = meta[1, :G]
    y = _ffn_call(gids, act, xs, ws, W_in, b_in, W_out, b_out)
    out = _combine_kernel()(y, pos0, pos1)
    return out
